# baseline (speedup 1.0000x reference)
import jax
import jax.numpy as jnp
from jax import lax
from jax.experimental import pallas as pl
from jax.experimental.pallas import tpu as pltpu
from jax.experimental.pallas import tpu_sc as plsc

ROWS = 32768
E = 64
K = 8
L = 16
NC = 2
NS = 16
NW = NC * NS
RPW = ROWS // NW
C = 256


def _tec_body(x_hbm, idx_hbm, soft_hbm, hard_hbm, x_v, idxo_v, hard_v):
  wid = lax.axis_index("s") * NC + lax.axis_index("c")
  iot = lax.iota(jnp.int32, L)
  m8 = iot < K
  perm8 = (iot + K) & (L - 1)
  zero16 = jnp.zeros((L,), jnp.int32)
  last16 = jnp.full((L,), L - 1, jnp.int32)

  def merge(a, b):
    ka, va = a
    kb, vb = b
    gk = jnp.take_along_axis(kb, perm8, axis=0)
    gv = jnp.take_along_axis(vb, perm8, axis=0)
    ck = jnp.where(m8, ka, gk)
    cv = jnp.where(m8, va, gv)
    return plsc.sort_key_val(ck, cv, descending=True)

  def chunk_body(ch, carry):
    row0 = wid * RPW + ch * C
    pltpu.sync_copy(x_hbm.at[pl.ds(row0, C)], x_v)

    @plsc.parallel_loop(0, C, unroll=2)
    def row_body(r):
      vals = []
      runs = []
      for j in range(E // L):
        v = x_v[r, pl.ds(j * L, L)]
        runs.append(plsc.sort_key_val(v, iot + jnp.int32(j * L),
                                      descending=True))
        vals.append(v)
      fv, fi = merge(merge(runs[0], runs[1]), merge(runs[2], runs[3]))
      mx = jnp.take_along_axis(fv, zero16, axis=0)

      es = [jnp.exp(v - mx) for v in vals]
      cs = jnp.cumsum(es[0] + es[1] + es[2] + es[3])
      sinv = 1.0 / jnp.take_along_axis(cs, last16, axis=0)
      for j in range(E // L):
        x_v[r, pl.ds(j * L, L)] = es[j] * sinv

      he = jnp.exp(fv - mx)
      hcs = jnp.cumsum(jnp.where(m8, he, 0.0))
      hinv = 1.0 / jnp.take_along_axis(hcs, last16, axis=0)
      rsplat = zero16 + r
      plsc.store_scatter(idxo_v, [rsplat, iot], fi, mask=m8)
      plsc.store_scatter(hard_v, [rsplat, iot], he * hinv, mask=m8)

    pltpu.sync_copy(x_v, soft_hbm.at[pl.ds(row0, C)])
    pltpu.sync_copy(idxo_v, idx_hbm.at[pl.ds(row0, C)])
    pltpu.sync_copy(hard_v, hard_hbm.at[pl.ds(row0, C)])
    return carry

  lax.fori_loop(0, RPW // C, chunk_body, 0)


@jax.jit
def _gate(x):
  mesh = plsc.VectorSubcoreMesh(
      core_axis_name="c", subcore_axis_name="s", num_cores=NC, num_subcores=NS
  )
  run = pl.kernel(
      _tec_body,
      out_type=(
          jax.ShapeDtypeStruct((ROWS, K), jnp.int32),
          jax.ShapeDtypeStruct((ROWS, E), jnp.float32),
          jax.ShapeDtypeStruct((ROWS, K), jnp.float32),
      ),
      mesh=mesh,
      compiler_params=pltpu.CompilerParams(
          needs_layout_passes=False, skip_device_barrier=True
      ),
      scratch_types=[
          pltpu.VMEM((C, E), jnp.float32),
          pltpu.VMEM((C, K), jnp.int32),
          pltpu.VMEM((C, K), jnp.float32),
      ],
  )
  return run(x)


def kernel(logits):
  return _gate(logits)


# speedup vs baseline: 1.0663x; 1.0663x over previous
import jax
import jax.numpy as jnp
from jax import lax
from jax.experimental import pallas as pl
from jax.experimental.pallas import tpu as pltpu
from jax.experimental.pallas import tpu_sc as plsc

ROWS = 32768
E = 64
K = 8
L = 16
NC = 2
NS = 16
NW = NC * NS
RPW = ROWS // NW
C = 128


def _tec_body(x_hbm, idx_hbm, soft_hbm, hard_hbm,
              x_v0, x_v1, idxo_v, hard_v, sem0, sem1):
  wid = lax.axis_index("s") * NC + lax.axis_index("c")
  iot = lax.iota(jnp.int32, L)
  m8 = iot < K
  perm8 = (iot + K) & (L - 1)
  zero16 = jnp.zeros((L,), jnp.int32)
  last16 = jnp.full((L,), L - 1, jnp.int32)

  def merge(a, b):
    ka, va = a
    kb, vb = b
    gk = jnp.take_along_axis(kb, perm8, axis=0)
    gv = jnp.take_along_axis(vb, perm8, axis=0)
    ck = jnp.where(m8, ka, gk)
    cv = jnp.where(m8, va, gv)
    return plsc.sort_key_val(ck, cv, descending=True)

  def do_chunk(row0, buf):
    @plsc.parallel_loop(0, C, unroll=2)
    def row_body(r):
      vals = []
      runs = []
      for j in range(E // L):
        v = buf[r, pl.ds(j * L, L)]
        runs.append(plsc.sort_key_val(v, iot + jnp.int32(j * L),
                                      descending=True))
        vals.append(v)
      fv, fi = merge(merge(runs[0], runs[1]), merge(runs[2], runs[3]))
      mx = jnp.take_along_axis(fv, zero16, axis=0)

      es = [jnp.exp(v - mx) for v in vals]
      cs = jnp.cumsum(es[0] + es[1] + es[2] + es[3])
      sinv = 1.0 / jnp.take_along_axis(cs, last16, axis=0)
      for j in range(E // L):
        buf[r, pl.ds(j * L, L)] = es[j] * sinv

      he = jnp.exp(fv - mx)
      hcs = jnp.cumsum(jnp.where(m8, he, 0.0))
      hinv = 1.0 / jnp.take_along_axis(hcs, last16, axis=0)
      rsplat = zero16 + r
      plsc.store_scatter(idxo_v, [rsplat, iot], fi, mask=m8)
      plsc.store_scatter(hard_v, [rsplat, iot], he * hinv, mask=m8)

    pltpu.sync_copy(buf, soft_hbm.at[pl.ds(row0, C)])
    pltpu.sync_copy(idxo_v, idx_hbm.at[pl.ds(row0, C)])
    pltpu.sync_copy(hard_v, hard_hbm.at[pl.ds(row0, C)])

  base0 = wid * RPW
  pltpu.async_copy(x_hbm.at[pl.ds(base0, C)], x_v0, sem0)
  nch2 = RPW // C // 2

  def chunk2_body(i, carry):
    base = base0 + i * (2 * C)
    pltpu.make_async_copy(x_hbm.at[pl.ds(base, C)], x_v0, sem0).wait()
    h1 = pltpu.async_copy(x_hbm.at[pl.ds(base + C, C)], x_v1, sem1)
    do_chunk(base, x_v0)

    @pl.when(i < nch2 - 1)
    def _():
      pltpu.async_copy(x_hbm.at[pl.ds(base + 2 * C, C)], x_v0, sem0)

    h1.wait()
    do_chunk(base + C, x_v1)
    return carry

  lax.fori_loop(0, nch2, chunk2_body, 0)


@jax.jit
def _gate(x):
  mesh = plsc.VectorSubcoreMesh(
      core_axis_name="c", subcore_axis_name="s", num_cores=NC, num_subcores=NS
  )
  run = pl.kernel(
      _tec_body,
      out_type=(
          jax.ShapeDtypeStruct((ROWS, K), jnp.int32),
          jax.ShapeDtypeStruct((ROWS, E), jnp.float32),
          jax.ShapeDtypeStruct((ROWS, K), jnp.float32),
      ),
      mesh=mesh,
      compiler_params=pltpu.CompilerParams(needs_layout_passes=False),
      scratch_types=[
          pltpu.VMEM((C, E), jnp.float32),
          pltpu.VMEM((C, E), jnp.float32),
          pltpu.VMEM((C, K), jnp.int32),
          pltpu.VMEM((C, K), jnp.float32),
          pltpu.SemaphoreType.DMA,
          pltpu.SemaphoreType.DMA,
      ],
  )
  return run(x)


def kernel(logits):
  return _gate(logits)
